# baseline (speedup 1.0000x reference)
import functools

import jax
import jax.numpy as jnp
from jax import lax
from jax.experimental import pallas as pl
from jax.experimental.pallas import tpu as pltpu
from jax.experimental.pallas import tpu_sc as plsc

_NUM_MODELS = 2
_NUM_BOXES = 1000000
_DIM = 32
_BATCH = 16384

_NC = 2
_NS = 16
_NW = _NC * _NS
_CHUNK = _BATCH // _NW
_GW = 128
_NG = _CHUNK // _GW

_mesh = plsc.VectorSubcoreMesh(core_axis_name="c", subcore_axis_name="s")


@functools.partial(
    pl.kernel,
    mesh=_mesh,
    compiler_params=pltpu.CompilerParams(use_tc_tiling_on_sc=False),
    out_type=jax.ShapeDtypeStruct((_NUM_MODELS, _BATCH, 2, _DIM), jnp.float32),
    scratch_types=[
        pltpu.VMEM((_NG, _GW), jnp.int32),
        pltpu.VMEM((_CHUNK, _DIM), jnp.float32),
        pltpu.VMEM((_CHUNK, _DIM), jnp.float32),
        pltpu.VMEM((_CHUNK, 2, _DIM), jnp.float32),
        pltpu.SemaphoreType.DMA,
    ],
)
def _deltaboxes_sc(ids_hbm, z0, z1, ld0, ld1, out_hbm,
                   idx_v, zrows, ldrows, obuf, sem):
    wid = lax.axis_index("s") * _NC + lax.axis_index("c")
    base = wid * _CHUNK
    pltpu.sync_copy(ids_hbm.at[pl.ds(wid * _NG, _NG)], idx_v)

    for m, (zt, ldt) in enumerate(((z0, ld0), (z1, ld1))):
        copies = []
        for j in range(_NG):
            copies.append(pltpu.async_copy(
                zt.at[idx_v.at[j]], zrows.at[pl.ds(j * _GW, _GW)], sem))
            copies.append(pltpu.async_copy(
                ldt.at[idx_v.at[j]], ldrows.at[pl.ds(j * _GW, _GW)], sem))
        for c in copies:
            c.wait()

        def body(b, carry):
            for k in range(_DIM // 16):
                zs = zrows[b, pl.ds(k * 16, 16)]
                ls = ldrows[b, pl.ds(k * 16, 16)]
                obuf[b, 0, pl.ds(k * 16, 16)] = zs
                obuf[b, 1, pl.ds(k * 16, 16)] = zs + jnp.exp(ls)
            return carry

        lax.fori_loop(0, _CHUNK, body, 0, unroll=4)
        pltpu.sync_copy(obuf, out_hbm.at[m, pl.ds(base, _CHUNK)])


def kernel(ids, z, logdelta):
    ids2 = ids.astype(jnp.int32).reshape(_NW * _NG, _GW)
    return _deltaboxes_sc(ids2, z[0], z[1], logdelta[0], logdelta[1])
